# baseline (speedup 1.0000x reference)
import functools

import jax
import jax.numpy as jnp
from jax import lax
from jax.experimental import pallas as pl
from jax.experimental.pallas import tpu as pltpu
from jax.experimental.pallas import tpu_sc as plsc

DIM = 512
NSL = 8
ROWS = 6
BATCH = 16384
NC, NSUB, L = 2, 16, 16
NW = NC * NSUB
BPW = BATCH // NW
CH = 32
NCHUNK = BPW // CH
NPAIR = NCHUNK // 2


def _mesh():
    return plsc.VectorSubcoreMesh(core_axis_name="c", subcore_axis_name="s")


@functools.partial(
    pl.kernel,
    out_type=jax.ShapeDtypeStruct((ROWS, 1, DIM), jnp.float32),
    mesh=_mesh(),
    scratch_types=[
        pltpu.VMEM((DIM * NSL,), jnp.float32),
        pltpu.VMEM((DIM,), jnp.float32),
    ],
)
def _slider_sum(wt_hbm, ws_hbm, wrow_v, acc_v):
    wid = lax.axis_index("s") * NC + lax.axis_index("c")

    @pl.when(wid < ROWS)
    def _():
        pltpu.sync_copy(wt_hbm.at[wid], wrow_v)

        def body(dv, carry):
            o = dv * L
            acc = wrow_v[pl.ds(o, L)]
            for i in range(1, NSL):
                acc = acc + wrow_v[pl.ds(i * DIM + o, L)]
            acc_v[pl.ds(o, L)] = acc
            return carry

        lax.fori_loop(0, DIM // L, body, 0)
        pltpu.sync_copy(acc_v, ws_hbm.at[wid, 0])


@functools.partial(
    pl.kernel,
    out_type=jax.ShapeDtypeStruct((BATCH, 1, DIM), jnp.float32),
    mesh=_mesh(),
    scratch_types=[
        pltpu.VMEM((BPW,), jnp.int32),
        pltpu.VMEM((ROWS, 1, DIM), jnp.float32),
        pltpu.VMEM((CH, 1, DIM), jnp.float32),
        pltpu.VMEM((CH, 1, DIM), jnp.float32),
        pltpu.VMEM((CH, 1, DIM), jnp.float32),
        pltpu.VMEM((CH, 1, DIM), jnp.float32),
        pltpu.SemaphoreType.DMA,
        pltpu.SemaphoreType.DMA,
        pltpu.SemaphoreType.DMA,
        pltpu.SemaphoreType.DMA,
    ],
)
def _walk_embed(z_hbm, idx_hbm, ws_hbm, out_hbm, idx_v, wsv,
                zb0, ob0, zb1, ob1,
                zs0, os0, zs1, os1):
    wid = lax.axis_index("s") * NC + lax.axis_index("c")
    base = wid * BPW

    zb, ob = (zb0, zb1), (ob0, ob1)
    zs, osm = (zs0, zs1), (os0, os1)

    def start_in(c, b):
        pltpu.async_copy(z_hbm.at[pl.ds(base + c * CH, CH)], zb[b], zs[b])

    start_in(0, 0)
    start_in(1, 1)
    pltpu.sync_copy(ws_hbm, wsv)
    pltpu.sync_copy(idx_hbm.at[pl.ds(base, BPW)], idx_v)

    def pair(it, carry):
        for b in range(2):
            c = it * 2 + b
            row0 = base + c * CH
            pltpu.make_async_copy(z_hbm.at[pl.ds(row0, CH)], zb[b], zs[b]).wait()

            @pl.when(it >= 1)
            def _():
                pltpu.make_async_copy(
                    ob[b], out_hbm.at[pl.ds(row0, CH)], osm[b]).wait()

            def group(g, c2):
                idxv = idx_v[pl.ds(c * CH + g * L, L)]
                for j in range(L):
                    s = idxv[j]
                    for v in range(DIM // L):
                        o = v * L
                        r = g * L + j
                        ob[b][r, 0, pl.ds(o, L)] = (
                            zb[b][r, 0, pl.ds(o, L)] + wsv[s, 0, pl.ds(o, L)])
                return c2

            lax.fori_loop(0, CH // L, group, 0)
            pltpu.async_copy(ob[b], out_hbm.at[pl.ds(row0, CH)], osm[b])

            @pl.when(it < NPAIR - 1)
            def _():
                start_in(c + 2, b)
        return carry

    lax.fori_loop(0, NPAIR, pair, 0)

    for b in range(2):
        row0 = base + (NCHUNK - 2 + b) * CH
        pltpu.make_async_copy(ob[b], out_hbm.at[pl.ds(row0, CH)], osm[b]).wait()


def kernel(z, w, index_, alpha=1):
    wt = jnp.transpose(w.reshape(ROWS, DIM, NSL), (0, 2, 1)).reshape(ROWS, NSL * DIM)
    ws = _slider_sum(wt)
    return _walk_embed(z, index_, ws)


# speedup vs baseline: 2.5240x; 2.5240x over previous
import functools

import jax
import jax.numpy as jnp
from jax import lax
from jax.experimental import pallas as pl
from jax.experimental.pallas import tpu as pltpu
from jax.experimental.pallas import tpu_sc as plsc

DIM = 512
NSL = 8
ROWS = 6
BATCH = 16384
NC, NSUB, L = 2, 16, 16
NW = NC * NSUB
BPW = BATCH // NW
CH = 32
NCHUNK = BPW // CH
NPAIR = NCHUNK // 2


def _mesh():
    return plsc.VectorSubcoreMesh(core_axis_name="c", subcore_axis_name="s")


@functools.partial(
    pl.kernel,
    out_type=jax.ShapeDtypeStruct((ROWS, 1, DIM), jnp.float32),
    mesh=_mesh(),
    scratch_types=[
        pltpu.VMEM((DIM * NSL,), jnp.float32),
        pltpu.VMEM((DIM,), jnp.float32),
    ],
)
def _slider_sum(wt_hbm, ws_hbm, wrow_v, acc_v):
    wid = lax.axis_index("s") * NC + lax.axis_index("c")

    @pl.when(wid < ROWS)
    def _():
        pltpu.sync_copy(wt_hbm.at[wid], wrow_v)

        def body(dv, carry):
            o = dv * L
            acc = wrow_v[pl.ds(o, L)]
            for i in range(1, NSL):
                acc = acc + wrow_v[pl.ds(i * DIM + o, L)]
            acc_v[pl.ds(o, L)] = acc
            return carry

        lax.fori_loop(0, DIM // L, body, 0)
        pltpu.sync_copy(acc_v, ws_hbm.at[wid, 0])


@functools.partial(
    pl.kernel,
    out_type=jax.ShapeDtypeStruct((BATCH, 1, DIM), jnp.float32),
    mesh=_mesh(),
    scratch_types=[
        pltpu.VMEM((BPW,), jnp.int32),
        pltpu.VMEM((ROWS, 1, DIM), jnp.float32),
        pltpu.VMEM((CH, 1, DIM), jnp.float32),
        pltpu.VMEM((CH, 1, DIM), jnp.float32),
        pltpu.VMEM((CH, 1, DIM), jnp.float32),
        pltpu.VMEM((CH, 1, DIM), jnp.float32),
        pltpu.SemaphoreType.DMA,
        pltpu.SemaphoreType.DMA,
        pltpu.SemaphoreType.DMA,
        pltpu.SemaphoreType.DMA,
    ],
)
def _walk_embed(z_hbm, idx_hbm, ws_hbm, out_hbm, idx_v, wsv,
                zb0, ob0, zb1, ob1,
                zs0, os0, zs1, os1):
    wid = lax.axis_index("s") * NC + lax.axis_index("c")
    base = wid * BPW

    zb, ob = (zb0, zb1), (ob0, ob1)
    zs, osm = (zs0, zs1), (os0, os1)

    def start_in(c, b):
        pltpu.async_copy(z_hbm.at[pl.ds(base + c * CH, CH)], zb[b], zs[b])

    start_in(0, 0)
    start_in(1, 1)
    pltpu.sync_copy(ws_hbm, wsv)
    pltpu.sync_copy(idx_hbm.at[pl.ds(base, BPW)], idx_v)

    def pair(it, carry):
        for b in range(2):
            c = it * 2 + b
            row0 = base + c * CH
            pltpu.make_async_copy(z_hbm.at[pl.ds(row0, CH)], zb[b], zs[b]).wait()

            @pl.when(it >= 1)
            def _():
                pltpu.make_async_copy(
                    ob[b], out_hbm.at[pl.ds(row0, CH)], osm[b]).wait()

            for g in range(CH // L):
                idxv = idx_v[pl.ds(c * CH + g * L, L)]
                svals = [idxv[j] for j in range(L)]

                @plsc.parallel_loop(0, DIM // L, 1, unroll=2)
                def _(v):
                    o = v * L
                    for j in range(L):
                        r = g * L + j
                        ob[b][r, 0, pl.ds(o, L)] = (
                            zb[b][r, 0, pl.ds(o, L)]
                            + wsv[svals[j], 0, pl.ds(o, L)])
            pltpu.async_copy(ob[b], out_hbm.at[pl.ds(row0, CH)], osm[b])

            @pl.when(it < NPAIR - 1)
            def _():
                start_in(c + 2, b)
        return carry

    lax.fori_loop(0, NPAIR, pair, 0)

    for b in range(2):
        row0 = base + (NCHUNK - 2 + b) * CH
        pltpu.make_async_copy(ob[b], out_hbm.at[pl.ds(row0, CH)], osm[b]).wait()


def kernel(z, w, index_, alpha=1):
    wt = jnp.transpose(w.reshape(ROWS, DIM, NSL), (0, 2, 1)).reshape(ROWS, NSL * DIM)
    ws = _slider_sum(wt)
    return _walk_embed(z, index_, ws)
